# baseline (speedup 1.0000x reference)
import functools

import jax
import jax.numpy as jnp
from jax import lax
from jax.experimental import pallas as pl
from jax.experimental.pallas import tpu as pltpu
from jax.experimental.pallas import tpu_sc as plsc

N = 4096
F = 512
D = 256
B = 8192

ROW_BLK = 1024


def _mlp_body(feat_ref, wemb_ref, bemb_ref, w0_ref, x1_ref):
    e = jnp.dot(feat_ref[...], wemb_ref[...], preferred_element_type=jnp.float32)
    e = jnp.maximum(e + bemb_ref[...], 0.0)
    x1 = jnp.dot(e, w0_ref[...], preferred_element_type=jnp.float32)
    x1_ref[...] = jnp.maximum(x1, 0.0)


def _stage2_body(adj_ref, x1_ref, w1_ref, t1_ref, x2_ref):
    t1 = jnp.dot(adj_ref[...], x1_ref[...], preferred_element_type=jnp.float32)
    t1_ref[...] = t1
    x2 = jnp.dot(t1, w1_ref[...], preferred_element_type=jnp.float32)
    x2_ref[...] = jnp.maximum(x2, 0.0)


def _stage3_body(adj_ref, t1f_ref, x1_ref, x2_ref, w2_ref, bcheb_ref, il_ref):
    t2 = 2.0 * jnp.dot(adj_ref[...], t1f_ref[...], preferred_element_type=jnp.float32)
    t2 = t2 - x1_ref[...]
    x3 = jnp.maximum(jnp.dot(t2, w2_ref[...], preferred_element_type=jnp.float32), 0.0)
    il_ref[...] = x1_ref[...] + x2_ref[...] + x3 + bcheb_ref[...]


def _loss_body(s_ref, loss_ref, wins_ref):
    pos = s_ref[0:1, :]
    neg = s_ref[1:2, :]
    diff = pos - neg
    sig = 1.0 / (1.0 + jnp.exp(-diff))
    loss_ref[0, 0] = jnp.sum(jnp.log(sig + 1e-9))
    wins_ref[0, 0] = jnp.sum((diff >= 0.0).astype(jnp.float32))


_PW = B // 32
_CH = 64
_NCH = _PW // _CH


def _sc_scores(table, idx_flat):
    info = plsc.get_sparse_core_info()
    nc = info.num_cores
    mesh = plsc.VectorSubcoreMesh(core_axis_name="c", subcore_axis_name="s")

    @functools.partial(
        pl.kernel,
        mesh=mesh,
        out_type=jax.ShapeDtypeStruct((2, B), jnp.float32),
        scratch_types=[
            pltpu.VMEM((3 * _PW,), jnp.int32),
            pltpu.VMEM((2, 3, _CH, D), jnp.float32),
            pltpu.VMEM((_PW,), jnp.float32),
            pltpu.VMEM((_PW,), jnp.float32),
            pltpu.SemaphoreType.DMA,
            pltpu.SemaphoreType.DMA,
        ],
    )
    def k(table_hbm, idx_hbm, out_hbm, idx_v, rows_v, ps_v, ns_v, sem0, sem1):
        wid = lax.axis_index("s") * nc + lax.axis_index("c")
        base = wid * _PW
        for t in range(3):
            pltpu.sync_copy(
                idx_hbm.at[pl.ds(t * B + base, _PW)],
                idx_v.at[pl.ds(t * _PW, _PW)],
            )
        sems = (sem0, sem1)

        def fire(c, buf):
            return [
                pltpu.async_copy(
                    table_hbm.at[idx_v.at[pl.ds(t * _PW + c * _CH, _CH)]],
                    rows_v.at[buf, t],
                    sems[buf],
                )
                for t in range(3)
            ]

        lane = lax.broadcasted_iota(jnp.int32, (16,), 0)
        dnums = lax.GatherDimensionNumbers(
            offset_dims=(), collapsed_slice_dims=(0,), start_index_map=(0,)
        )

        def shuffle(x, perm):
            return lax.gather(
                x, perm[:, None], dnums, slice_sizes=(1,),
                mode=lax.GatherScatterMode.PROMISE_IN_BOUNDS,
            )

        def compute(c, buf):
            def group_fn(g, _):
                def row_fn(r, carry):
                    pv, nv = carry
                    row = g * 16 + r
                    accp = jnp.zeros((16,), jnp.float32)
                    accn = jnp.zeros((16,), jnp.float32)
                    for j in range(D // 16):
                        kv = rows_v[buf, 0, row, pl.ds(j * 16, 16)]
                        accp = accp + kv * rows_v[buf, 1, row, pl.ds(j * 16, 16)]
                        accn = accn + kv * rows_v[buf, 2, row, pl.ds(j * 16, 16)]
                    for s in (8, 4, 2, 1):
                        perm = lane ^ s
                        accp = accp + shuffle(accp, perm)
                        accn = accn + shuffle(accn, perm)
                    pv = jnp.where(lane == r, accp, pv)
                    nv = jnp.where(lane == r, accn, nv)
                    return (pv, nv)

                pv, nv = lax.fori_loop(
                    0, 16, row_fn,
                    (jnp.zeros((16,), jnp.float32), jnp.zeros((16,), jnp.float32)),
                )
                ps_v[pl.ds(c * _CH + g * 16, 16)] = pv
                ns_v[pl.ds(c * _CH + g * 16, 16)] = nv
                return _

            lax.fori_loop(0, _CH // 16, group_fn, 0)

        handles = {0: fire(0, 0)}
        for c in range(_NCH):
            if c + 1 < _NCH:
                handles[c + 1] = fire(c + 1, (c + 1) % 2)
            for h in handles[c]:
                h.wait()
            compute(c, c % 2)

        pltpu.sync_copy(ps_v, out_hbm.at[0, pl.ds(base, _PW)])
        pltpu.sync_copy(ns_v, out_hbm.at[1, pl.ds(base, _PW)])

    return k(table, idx_flat)


def kernel(features, adj, train_set, epoch, W_emb, b_emb, W_cheb, b_cheb):
    del epoch
    n_blk = N // ROW_BLK
    bemb2 = b_emb.reshape(1, D)
    bcheb2 = b_cheb.reshape(1, D)

    x1 = pl.pallas_call(
        _mlp_body,
        grid=(n_blk,),
        in_specs=[
            pl.BlockSpec((ROW_BLK, F), lambda i: (i, 0)),
            pl.BlockSpec((F, D), lambda i: (0, 0)),
            pl.BlockSpec((1, D), lambda i: (0, 0)),
            pl.BlockSpec((D, D), lambda i: (0, 0)),
        ],
        out_specs=pl.BlockSpec((ROW_BLK, D), lambda i: (i, 0)),
        out_shape=jax.ShapeDtypeStruct((N, D), jnp.float32),
    )(features, W_emb, bemb2, W_cheb[0])

    t1, x2 = pl.pallas_call(
        _stage2_body,
        grid=(n_blk,),
        in_specs=[
            pl.BlockSpec((ROW_BLK, N), lambda i: (i, 0)),
            pl.BlockSpec((N, D), lambda i: (0, 0)),
            pl.BlockSpec((D, D), lambda i: (0, 0)),
        ],
        out_specs=[
            pl.BlockSpec((ROW_BLK, D), lambda i: (i, 0)),
            pl.BlockSpec((ROW_BLK, D), lambda i: (i, 0)),
        ],
        out_shape=[
            jax.ShapeDtypeStruct((N, D), jnp.float32),
            jax.ShapeDtypeStruct((N, D), jnp.float32),
        ],
    )(adj, x1, W_cheb[1])

    item_latent = pl.pallas_call(
        _stage3_body,
        grid=(n_blk,),
        in_specs=[
            pl.BlockSpec((ROW_BLK, N), lambda i: (i, 0)),
            pl.BlockSpec((N, D), lambda i: (0, 0)),
            pl.BlockSpec((ROW_BLK, D), lambda i: (i, 0)),
            pl.BlockSpec((ROW_BLK, D), lambda i: (i, 0)),
            pl.BlockSpec((D, D), lambda i: (0, 0)),
            pl.BlockSpec((1, D), lambda i: (0, 0)),
        ],
        out_specs=pl.BlockSpec((ROW_BLK, D), lambda i: (i, 0)),
        out_shape=jax.ShapeDtypeStruct((N, D), jnp.float32),
    )(adj, t1, x1, x2, W_cheb[2], bcheb2)

    idx_flat = jnp.concatenate(
        [train_set[:, 0], train_set[:, 1], train_set[:, 2]], axis=0
    )
    scores = _sc_scores(item_latent, idx_flat)

    loss_sum, wins = pl.pallas_call(
        _loss_body,
        grid=(1,),
        in_specs=[pl.BlockSpec((2, B), lambda i: (0, 0))],
        out_specs=[
            pl.BlockSpec(memory_space=pltpu.SMEM),
            pl.BlockSpec(memory_space=pltpu.SMEM),
        ],
        out_shape=[
            jax.ShapeDtypeStruct((1, 1), jnp.float32),
            jax.ShapeDtypeStruct((1, 1), jnp.float32),
        ],
    )(scores)

    bf = jnp.float32(B)
    wins_s = wins[0, 0]
    loss = -(loss_sum[0, 0] / bf)
    hr = wins_s / bf
    mrr = (wins_s * jnp.float32(1e-9) + (bf - wins_s)) / bf
    ndcg = (wins_s + (bf - wins_s) * jnp.float32(2.0 / 3.0)) / bf
    return (loss, mrr, hr, ndcg)


# speedup vs baseline: 1.2060x; 1.2060x over previous
import functools

import jax
import jax.numpy as jnp
from jax import lax
from jax.experimental import pallas as pl
from jax.experimental.pallas import tpu as pltpu
from jax.experimental.pallas import tpu_sc as plsc

N = 4096
F = 512
D = 256
B = 8192

_FB = 1024
_AB = 256
_CB = 512


def _dense_body(feat_hbm, adj_hbm, wemb_ref, bemb_ref, w0_ref, w1_ref,
                w2_ref, bcheb_ref, il_ref,
                f0, f1, a0, a1, adjbf, x1s, t1bf,
                fsem0, fsem1, asem0, asem1):
    fbufs, fsems = (f0, f1), (fsem0, fsem1)
    abufs, asems = (a0, a1), (asem0, asem1)

    def fcopy(i, b):
        return pltpu.make_async_copy(
            feat_hbm.at[pl.ds(i * _FB, _FB), :], fbufs[b], fsems[b])

    def acopy(i, b):
        return pltpu.make_async_copy(
            adj_hbm.at[pl.ds(i * _AB, _AB), :], abufs[b], asems[b])

    nf = N // _FB
    na = N // _AB

    def a_compute(c, b):
        fcopy(c, b).wait()
        e = jnp.dot(fbufs[b][...], wemb_ref[...],
                    preferred_element_type=jnp.float32)
        e = jnp.maximum(e + bemb_ref[...], 0.0)
        x1b = jnp.dot(e, w0_ref[...], preferred_element_type=jnp.float32)
        x1s[pl.ds(c * _FB, _FB), :] = jnp.maximum(x1b, 0.0)

    def b_compute(c, b):
        acopy(c, b).wait()
        ablk = abufs[b][...]
        t1b = jnp.dot(ablk, x1s[...], preferred_element_type=jnp.float32)
        t1bf[pl.ds(c * _AB, _AB), :] = t1b.astype(jnp.bfloat16)
        adjbf[pl.ds(c * _AB, _AB), :] = ablk.astype(jnp.bfloat16)

    fcopy(0, 0).start()
    fcopy(1, 1).start()

    def a_pair(i, carry):
        c = 2 * i
        a_compute(c, 0)
        fcopy(c + 2, 0).start()
        a_compute(c + 1, 1)
        fcopy(c + 3, 1).start()
        return carry

    lax.fori_loop(0, nf // 2 - 1, a_pair, 0)
    a_compute(nf - 2, 0)
    acopy(0, 0).start()
    acopy(1, 1).start()
    a_compute(nf - 1, 1)

    def b_pair(i, carry):
        c = 2 * i
        b_compute(c, 0)
        acopy(c + 2, 0).start()
        b_compute(c + 1, 1)
        acopy(c + 3, 1).start()
        return carry

    lax.fori_loop(0, na // 2 - 1, b_pair, 0)
    b_compute(na - 2, 0)
    b_compute(na - 1, 1)

    def c_block(i, carry):
        sl = pl.ds(i * _CB, _CB)
        x1b = x1s[sl, :]
        t2 = 2.0 * jnp.dot(adjbf[sl, :], t1bf[...],
                           preferred_element_type=jnp.float32) - x1b
        x2 = jnp.maximum(
            jnp.dot(t1bf[sl, :], w1_ref[...],
                    preferred_element_type=jnp.float32), 0.0)
        x3 = jnp.maximum(
            jnp.dot(t2, w2_ref[...], preferred_element_type=jnp.float32), 0.0)
        il_ref[sl, :] = x1b + x2 + x3 + bcheb_ref[...]
        return carry

    lax.fori_loop(0, N // _CB, c_block, 0)


def _loss_body(s_ref, loss_ref, wins_ref):
    pos = s_ref[0:1, :]
    neg = s_ref[1:2, :]
    diff = pos - neg
    sig = 1.0 / (1.0 + jnp.exp(-diff))
    loss_ref[0, 0] = jnp.sum(jnp.log(sig + 1e-9))
    wins_ref[0, 0] = jnp.sum((diff >= 0.0).astype(jnp.float32))


_PW = B // 32
_CH = 64
_NCH = _PW // _CH


def _sc_scores(table, idx_flat):
    info = plsc.get_sparse_core_info()
    nc = info.num_cores
    mesh = plsc.VectorSubcoreMesh(core_axis_name="c", subcore_axis_name="s")

    @functools.partial(
        pl.kernel,
        mesh=mesh,
        out_type=jax.ShapeDtypeStruct((2, B), jnp.float32),
        scratch_types=[
            pltpu.VMEM((3 * _PW,), jnp.int32),
            pltpu.VMEM((2, 3, _CH, D), jnp.float32),
            pltpu.VMEM((_PW,), jnp.float32),
            pltpu.VMEM((_PW,), jnp.float32),
            pltpu.SemaphoreType.DMA,
            pltpu.SemaphoreType.DMA,
        ],
    )
    def k(table_hbm, idx_hbm, out_hbm, idx_v, rows_v, ps_v, ns_v, sem0, sem1):
        wid = lax.axis_index("s") * nc + lax.axis_index("c")
        base = wid * _PW
        for t in range(3):
            pltpu.sync_copy(
                idx_hbm.at[pl.ds(t * B + base, _PW)],
                idx_v.at[pl.ds(t * _PW, _PW)],
            )
        sems = (sem0, sem1)

        def fire(c, buf):
            return [
                pltpu.async_copy(
                    table_hbm.at[idx_v.at[pl.ds(t * _PW + c * _CH, _CH)]],
                    rows_v.at[buf, t],
                    sems[buf],
                )
                for t in range(3)
            ]

        lane = lax.broadcasted_iota(jnp.int32, (16,), 0)
        dnums = lax.GatherDimensionNumbers(
            offset_dims=(), collapsed_slice_dims=(0,), start_index_map=(0,)
        )

        def shuffle(x, perm):
            return lax.gather(
                x, perm[:, None], dnums, slice_sizes=(1,),
                mode=lax.GatherScatterMode.PROMISE_IN_BOUNDS,
            )

        def compute(c, buf):
            def group_fn(g, _):
                def row_fn(r, carry):
                    pv, nv = carry
                    row = g * 16 + r
                    accp = jnp.zeros((16,), jnp.float32)
                    accn = jnp.zeros((16,), jnp.float32)
                    for j in range(D // 16):
                        kv = rows_v[buf, 0, row, pl.ds(j * 16, 16)]
                        accp = accp + kv * rows_v[buf, 1, row, pl.ds(j * 16, 16)]
                        accn = accn + kv * rows_v[buf, 2, row, pl.ds(j * 16, 16)]
                    for s in (8, 4, 2, 1):
                        perm = lane ^ s
                        accp = accp + shuffle(accp, perm)
                        accn = accn + shuffle(accn, perm)
                    pv = jnp.where(lane == r, accp, pv)
                    nv = jnp.where(lane == r, accn, nv)
                    return (pv, nv)

                pv, nv = lax.fori_loop(
                    0, 16, row_fn,
                    (jnp.zeros((16,), jnp.float32), jnp.zeros((16,), jnp.float32)),
                )
                ps_v[pl.ds(c * _CH + g * 16, 16)] = pv
                ns_v[pl.ds(c * _CH + g * 16, 16)] = nv
                return _

            lax.fori_loop(0, _CH // 16, group_fn, 0)

        handles = {0: fire(0, 0)}
        for c in range(_NCH):
            if c + 1 < _NCH:
                handles[c + 1] = fire(c + 1, (c + 1) % 2)
            for h in handles[c]:
                h.wait()
            compute(c, c % 2)

        pltpu.sync_copy(ps_v, out_hbm.at[0, pl.ds(base, _PW)])
        pltpu.sync_copy(ns_v, out_hbm.at[1, pl.ds(base, _PW)])

    return k(table, idx_flat)


def kernel(features, adj, train_set, epoch, W_emb, b_emb, W_cheb, b_cheb):
    del epoch
    bemb2 = b_emb.reshape(1, D)
    bcheb2 = b_cheb.reshape(1, D)

    item_latent = pl.pallas_call(
        _dense_body,
        in_specs=[
            pl.BlockSpec(memory_space=pl.ANY),
            pl.BlockSpec(memory_space=pl.ANY),
            pl.BlockSpec(memory_space=pltpu.MemorySpace.VMEM),
            pl.BlockSpec(memory_space=pltpu.MemorySpace.VMEM),
            pl.BlockSpec(memory_space=pltpu.MemorySpace.VMEM),
            pl.BlockSpec(memory_space=pltpu.MemorySpace.VMEM),
            pl.BlockSpec(memory_space=pltpu.MemorySpace.VMEM),
            pl.BlockSpec(memory_space=pltpu.MemorySpace.VMEM),
        ],
        out_specs=pl.BlockSpec(memory_space=pltpu.MemorySpace.VMEM),
        out_shape=jax.ShapeDtypeStruct((N, D), jnp.float32),
        scratch_shapes=[
            pltpu.VMEM((_FB, F), jnp.float32),
            pltpu.VMEM((_FB, F), jnp.float32),
            pltpu.VMEM((_AB, N), jnp.float32),
            pltpu.VMEM((_AB, N), jnp.float32),
            pltpu.VMEM((N, N), jnp.bfloat16),
            pltpu.VMEM((N, D), jnp.float32),
            pltpu.VMEM((N, D), jnp.bfloat16),
            pltpu.SemaphoreType.DMA,
            pltpu.SemaphoreType.DMA,
            pltpu.SemaphoreType.DMA,
            pltpu.SemaphoreType.DMA,
        ],
    )(features, adj, W_emb, bemb2, W_cheb[0], W_cheb[1], W_cheb[2], bcheb2)

    idx_flat = jnp.concatenate(
        [train_set[:, 0], train_set[:, 1], train_set[:, 2]], axis=0
    )
    scores = _sc_scores(item_latent, idx_flat)

    loss_sum, wins = pl.pallas_call(
        _loss_body,
        grid=(1,),
        in_specs=[pl.BlockSpec((2, B), lambda i: (0, 0))],
        out_specs=[
            pl.BlockSpec(memory_space=pltpu.SMEM),
            pl.BlockSpec(memory_space=pltpu.SMEM),
        ],
        out_shape=[
            jax.ShapeDtypeStruct((1, 1), jnp.float32),
            jax.ShapeDtypeStruct((1, 1), jnp.float32),
        ],
    )(scores)

    bf = jnp.float32(B)
    wins_s = wins[0, 0]
    loss = -(loss_sum[0, 0] / bf)
    hr = wins_s / bf
    mrr = (wins_s * jnp.float32(1e-9) + (bf - wins_s)) / bf
    ndcg = (wins_s + (bf - wins_s) * jnp.float32(2.0 / 3.0)) / bf
    return (loss, mrr, hr, ndcg)


# speedup vs baseline: 1.2387x; 1.0271x over previous
import functools

import jax
import jax.numpy as jnp
from jax import lax
from jax.experimental import pallas as pl
from jax.experimental.pallas import tpu as pltpu
from jax.experimental.pallas import tpu_sc as plsc

N = 4096
F = 512
D = 256
B = 8192

_FB = 512
_AB = 256
_CB = 512


def _dense_body(feat_hbm, adj_hbm, wemb_ref, bemb_ref, w0_ref, w1_ref,
                w2_ref, bcheb_ref, il_ref,
                f0, f1, a0, a1, a2, adjbf, x1s, t1bf,
                fsem0, fsem1, asem0, asem1, asem2):
    fbufs, fsems = (f0, f1), (fsem0, fsem1)
    abufs, asems = (a0, a1, a2), (asem0, asem1, asem2)

    def fcopy(i, b):
        return pltpu.make_async_copy(
            feat_hbm.at[pl.ds(i * _FB, _FB), :], fbufs[b], fsems[b])

    def acopy(i, b):
        return pltpu.make_async_copy(
            adj_hbm.at[pl.ds(i * _AB, _AB), :], abufs[b], asems[b])

    nf = N // _FB
    na = N // _AB

    def a_compute(c, b):
        fcopy(c, b).wait()
        e = jnp.dot(fbufs[b][...], wemb_ref[...],
                    preferred_element_type=jnp.float32)
        e = jnp.maximum(e + bemb_ref[...], 0.0)
        x1b = jnp.dot(e, w0_ref[...], preferred_element_type=jnp.float32)
        x1s[pl.ds(c * _FB, _FB), :] = jnp.maximum(x1b, 0.0)

    def b_compute(c, b):
        acopy(c, b).wait()
        ablk = abufs[b][...]
        t1b = jnp.dot(ablk, x1s[...], preferred_element_type=jnp.float32)
        t1bf[pl.ds(c * _AB, _AB), :] = t1b.astype(jnp.bfloat16)
        adjbf[pl.ds(c * _AB, _AB), :] = ablk.astype(jnp.bfloat16)

    fcopy(0, 0).start()
    fcopy(1, 1).start()
    acopy(0, 0).start()
    acopy(1, 1).start()
    acopy(2, 2).start()

    def a_pair(i, carry):
        c = 2 * i
        a_compute(c, 0)
        fcopy(c + 2, 0).start()
        a_compute(c + 1, 1)
        fcopy(c + 3, 1).start()
        return carry

    lax.fori_loop(0, nf // 2 - 1, a_pair, 0)
    a_compute(nf - 2, 0)
    a_compute(nf - 1, 1)

    def b_trip(i, carry):
        c = 3 * i
        b_compute(c, 0)
        acopy(c + 3, 0).start()
        b_compute(c + 1, 1)
        acopy(c + 4, 1).start()
        b_compute(c + 2, 2)
        acopy(c + 5, 2).start()
        return carry

    lax.fori_loop(0, na // 3 - 1, b_trip, 0)
    b_compute(na - 4, 0)
    acopy(na - 1, 0).start()
    b_compute(na - 3, 1)
    b_compute(na - 2, 2)
    b_compute(na - 1, 0)

    def c_block(i, carry):
        sl = pl.ds(i * _CB, _CB)
        x1b = x1s[sl, :]
        t2 = 2.0 * jnp.dot(adjbf[sl, :], t1bf[...],
                           preferred_element_type=jnp.float32) - x1b
        x2 = jnp.maximum(
            jnp.dot(t1bf[sl, :], w1_ref[...],
                    preferred_element_type=jnp.float32), 0.0)
        x3 = jnp.maximum(
            jnp.dot(t2, w2_ref[...], preferred_element_type=jnp.float32), 0.0)
        il_ref[sl, :] = x1b + x2 + x3 + bcheb_ref[...]
        return carry

    lax.fori_loop(0, N // _CB, c_block, 0)


def _loss_body(s_ref, loss_ref, wins_ref):
    pos = s_ref[0:1, :]
    neg = s_ref[1:2, :]
    diff = pos - neg
    sig = 1.0 / (1.0 + jnp.exp(-diff))
    loss_ref[0, 0] = jnp.sum(jnp.log(sig + 1e-9))
    wins_ref[0, 0] = jnp.sum((diff >= 0.0).astype(jnp.float32))


_PW = B // 32
_CH = 64
_NCH = _PW // _CH


def _sc_scores(table, idx_flat):
    info = plsc.get_sparse_core_info()
    nc = info.num_cores
    mesh = plsc.VectorSubcoreMesh(core_axis_name="c", subcore_axis_name="s")

    @functools.partial(
        pl.kernel,
        mesh=mesh,
        out_type=jax.ShapeDtypeStruct((2, B), jnp.float32),
        scratch_types=[
            pltpu.VMEM((3 * _PW,), jnp.int32),
            pltpu.VMEM((2, 3, _CH, D), jnp.float32),
            pltpu.VMEM((_PW,), jnp.float32),
            pltpu.VMEM((_PW,), jnp.float32),
            pltpu.SemaphoreType.DMA,
            pltpu.SemaphoreType.DMA,
        ],
    )
    def k(table_hbm, idx_hbm, out_hbm, idx_v, rows_v, ps_v, ns_v, sem0, sem1):
        wid = lax.axis_index("s") * nc + lax.axis_index("c")
        base = wid * _PW
        for t in range(3):
            pltpu.sync_copy(
                idx_hbm.at[pl.ds(t * B + base, _PW)],
                idx_v.at[pl.ds(t * _PW, _PW)],
            )
        sems = (sem0, sem1)

        def fire(c, buf):
            return [
                pltpu.async_copy(
                    table_hbm.at[idx_v.at[pl.ds(t * _PW + c * _CH, _CH)]],
                    rows_v.at[buf, t],
                    sems[buf],
                )
                for t in range(3)
            ]

        lane = lax.broadcasted_iota(jnp.int32, (16,), 0)
        dnums = lax.GatherDimensionNumbers(
            offset_dims=(), collapsed_slice_dims=(0,), start_index_map=(0,)
        )

        def shuffle(x, perm):
            return lax.gather(
                x, perm[:, None], dnums, slice_sizes=(1,),
                mode=lax.GatherScatterMode.PROMISE_IN_BOUNDS,
            )

        def compute(c, buf):
            def group_fn(g, _):
                def row_fn(r, carry):
                    pv, nv = carry
                    row = g * 16 + r
                    accp = jnp.zeros((16,), jnp.float32)
                    accn = jnp.zeros((16,), jnp.float32)
                    for j in range(D // 16):
                        kv = rows_v[buf, 0, row, pl.ds(j * 16, 16)]
                        accp = accp + kv * rows_v[buf, 1, row, pl.ds(j * 16, 16)]
                        accn = accn + kv * rows_v[buf, 2, row, pl.ds(j * 16, 16)]
                    for s in (8, 4, 2, 1):
                        perm = lane ^ s
                        accp = accp + shuffle(accp, perm)
                        accn = accn + shuffle(accn, perm)
                    pv = jnp.where(lane == r, accp, pv)
                    nv = jnp.where(lane == r, accn, nv)
                    return (pv, nv)

                pv, nv = lax.fori_loop(
                    0, 16, row_fn,
                    (jnp.zeros((16,), jnp.float32), jnp.zeros((16,), jnp.float32)),
                )
                ps_v[pl.ds(c * _CH + g * 16, 16)] = pv
                ns_v[pl.ds(c * _CH + g * 16, 16)] = nv
                return _

            lax.fori_loop(0, _CH // 16, group_fn, 0)

        handles = {0: fire(0, 0)}
        for c in range(_NCH):
            if c + 1 < _NCH:
                handles[c + 1] = fire(c + 1, (c + 1) % 2)
            for h in handles[c]:
                h.wait()
            compute(c, c % 2)

        pltpu.sync_copy(ps_v, out_hbm.at[0, pl.ds(base, _PW)])
        pltpu.sync_copy(ns_v, out_hbm.at[1, pl.ds(base, _PW)])

    return k(table, idx_flat)


def kernel(features, adj, train_set, epoch, W_emb, b_emb, W_cheb, b_cheb):
    del epoch
    bemb2 = b_emb.reshape(1, D)
    bcheb2 = b_cheb.reshape(1, D)

    item_latent = pl.pallas_call(
        _dense_body,
        in_specs=[
            pl.BlockSpec(memory_space=pl.ANY),
            pl.BlockSpec(memory_space=pl.ANY),
            pl.BlockSpec(memory_space=pltpu.MemorySpace.VMEM),
            pl.BlockSpec(memory_space=pltpu.MemorySpace.VMEM),
            pl.BlockSpec(memory_space=pltpu.MemorySpace.VMEM),
            pl.BlockSpec(memory_space=pltpu.MemorySpace.VMEM),
            pl.BlockSpec(memory_space=pltpu.MemorySpace.VMEM),
            pl.BlockSpec(memory_space=pltpu.MemorySpace.VMEM),
        ],
        out_specs=pl.BlockSpec(memory_space=pltpu.MemorySpace.VMEM),
        out_shape=jax.ShapeDtypeStruct((N, D), jnp.float32),
        scratch_shapes=[
            pltpu.VMEM((_FB, F), jnp.float32),
            pltpu.VMEM((_FB, F), jnp.float32),
            pltpu.VMEM((_AB, N), jnp.float32),
            pltpu.VMEM((_AB, N), jnp.float32),
            pltpu.VMEM((_AB, N), jnp.float32),
            pltpu.VMEM((N, N), jnp.bfloat16),
            pltpu.VMEM((N, D), jnp.float32),
            pltpu.VMEM((N, D), jnp.bfloat16),
            pltpu.SemaphoreType.DMA,
            pltpu.SemaphoreType.DMA,
            pltpu.SemaphoreType.DMA,
            pltpu.SemaphoreType.DMA,
            pltpu.SemaphoreType.DMA,
        ],
    )(features, adj, W_emb, bemb2, W_cheb[0], W_cheb[1], W_cheb[2], bcheb2)

    idx_flat = jnp.concatenate(
        [train_set[:, 0], train_set[:, 1], train_set[:, 2]], axis=0
    )
    scores = _sc_scores(item_latent, idx_flat)

    loss_sum, wins = pl.pallas_call(
        _loss_body,
        grid=(1,),
        in_specs=[pl.BlockSpec((2, B), lambda i: (0, 0))],
        out_specs=[
            pl.BlockSpec(memory_space=pltpu.SMEM),
            pl.BlockSpec(memory_space=pltpu.SMEM),
        ],
        out_shape=[
            jax.ShapeDtypeStruct((1, 1), jnp.float32),
            jax.ShapeDtypeStruct((1, 1), jnp.float32),
        ],
    )(scores)

    bf = jnp.float32(B)
    wins_s = wins[0, 0]
    loss = -(loss_sum[0, 0] / bf)
    hr = wins_s / bf
    mrr = (wins_s * jnp.float32(1e-9) + (bf - wins_s)) / bf
    ndcg = (wins_s + (bf - wins_s) * jnp.float32(2.0 / 3.0)) / bf
    return (loss, mrr, hr, ndcg)


# speedup vs baseline: 1.2861x; 1.0383x over previous
import functools

import jax
import jax.numpy as jnp
from jax import lax
from jax.experimental import pallas as pl
from jax.experimental.pallas import tpu as pltpu
from jax.experimental.pallas import tpu_sc as plsc

N = 4096
F = 512
D = 256
B = 8192

_FB = 512
_AB = 256
_CB = 512


def _dense_body(feat_hbm, adj_hbm, wemb_ref, bemb_ref, w0_ref, w1_ref,
                w2_ref, bcheb_ref, il_ref,
                f0, f1, a0, a1, a2, adjbf, x1s, t1bf,
                fsem0, fsem1, asem0, asem1, asem2):
    fbufs, fsems = (f0, f1), (fsem0, fsem1)
    abufs, asems = (a0, a1, a2), (asem0, asem1, asem2)

    def fcopy(i, b):
        return pltpu.make_async_copy(
            feat_hbm.at[pl.ds(i * _FB, _FB), :], fbufs[b], fsems[b])

    def acopy(i, b):
        return pltpu.make_async_copy(
            adj_hbm.at[pl.ds(i * _AB, _AB), :], abufs[b], asems[b])

    nf = N // _FB
    na = N // _AB

    def a_compute(c, b):
        fcopy(c, b).wait()
        e = jnp.dot(fbufs[b][...], wemb_ref[...],
                    preferred_element_type=jnp.float32)
        e = jnp.maximum(e + bemb_ref[...], 0.0)
        x1b = jnp.dot(e, w0_ref[...], preferred_element_type=jnp.float32)
        x1s[pl.ds(c * _FB, _FB), :] = jnp.maximum(x1b, 0.0)

    def b_compute(c, b):
        acopy(c, b).wait()
        ablk = abufs[b][...]
        sl = pl.ds(c * _AB, _AB)
        t1b = jnp.dot(ablk, x1s[...], preferred_element_type=jnp.float32)
        t1bf[sl, :] = t1b.astype(jnp.bfloat16)
        adjbf[sl, :] = ablk.astype(jnp.bfloat16)
        x2b = jnp.maximum(
            jnp.dot(t1b.astype(jnp.bfloat16), w1_ref[...],
                    preferred_element_type=jnp.float32), 0.0)
        il_ref[sl, :] = x1s[sl, :] + x2b + bcheb_ref[...]

    fcopy(0, 0).start()
    fcopy(1, 1).start()
    acopy(0, 0).start()
    acopy(1, 1).start()
    acopy(2, 2).start()

    def a_pair(i, carry):
        c = 2 * i
        a_compute(c, 0)
        fcopy(c + 2, 0).start()
        a_compute(c + 1, 1)
        fcopy(c + 3, 1).start()
        return carry

    lax.fori_loop(0, nf // 2 - 1, a_pair, 0)
    a_compute(nf - 2, 0)
    a_compute(nf - 1, 1)

    def b_trip(i, carry):
        c = 3 * i
        b_compute(c, 0)
        acopy(c + 3, 0).start()
        b_compute(c + 1, 1)
        acopy(c + 4, 1).start()
        b_compute(c + 2, 2)
        acopy(c + 5, 2).start()
        return carry

    lax.fori_loop(0, na // 3 - 1, b_trip, 0)
    b_compute(na - 4, 0)
    acopy(na - 1, 0).start()
    b_compute(na - 3, 1)
    b_compute(na - 2, 2)
    b_compute(na - 1, 0)

    def c_block(i, carry):
        sl = pl.ds(i * _CB, _CB)
        t2 = 2.0 * jnp.dot(adjbf[sl, :], t1bf[...],
                           preferred_element_type=jnp.float32) - x1s[sl, :]
        x3 = jnp.maximum(
            jnp.dot(t2, w2_ref[...], preferred_element_type=jnp.float32), 0.0)
        il_ref[sl, :] = il_ref[sl, :] + x3
        return carry

    lax.fori_loop(0, N // _CB, c_block, 0)


def _loss_body(s_ref, loss_ref, mrr_ref, hr_ref, ndcg_ref):
    pos = s_ref[0:1, :]
    neg = s_ref[1:2, :]
    diff = pos - neg
    sig = 1.0 / (1.0 + jnp.exp(-diff))
    bf = jnp.float32(B)
    wins = jnp.sum((diff >= 0.0).astype(jnp.float32))
    loss_ref[0, 0] = -(jnp.sum(jnp.log(sig + 1e-9)) / bf)
    hr_ref[0, 0] = wins / bf
    mrr_ref[0, 0] = (wins * jnp.float32(1e-9) + (bf - wins)) / bf
    ndcg_ref[0, 0] = (wins + (bf - wins) * jnp.float32(2.0 / 3.0)) / bf


_PW = B // 32
_CH = 64
_NCH = _PW // _CH


def _sc_scores(table, ts_flat):
    info = plsc.get_sparse_core_info()
    nc = info.num_cores
    mesh = plsc.VectorSubcoreMesh(core_axis_name="c", subcore_axis_name="s")

    @functools.partial(
        pl.kernel,
        mesh=mesh,
        out_type=jax.ShapeDtypeStruct((2, B), jnp.float32),
        scratch_types=[
            pltpu.VMEM((3 * _PW,), jnp.int32),
            pltpu.VMEM((3 * _PW,), jnp.int32),
            pltpu.VMEM((2, 3, _CH, D), jnp.float32),
            pltpu.VMEM((_PW,), jnp.float32),
            pltpu.VMEM((_PW,), jnp.float32),
            pltpu.SemaphoreType.DMA,
            pltpu.SemaphoreType.DMA,
        ],
    )
    def k(table_hbm, ts_hbm, out_hbm, raw_v, idx_v, rows_v, ps_v, ns_v,
          sem0, sem1):
        wid = lax.axis_index("s") * nc + lax.axis_index("c")
        base = wid * _PW
        pltpu.sync_copy(ts_hbm.at[pl.ds(3 * base, 3 * _PW)], raw_v)
        dnums_i = lax.GatherDimensionNumbers(
            offset_dims=(), collapsed_slice_dims=(0,), start_index_map=(0,)
        )

        def shuf_i(x, perm):
            return lax.gather(
                x, perm[:, None], dnums_i, slice_sizes=(1,),
                mode=lax.GatherScatterMode.PROMISE_IN_BOUNDS,
            )

        lane_i = lax.broadcasted_iota(jnp.int32, (16,), 0)
        perms = [((lane_i * 3 + t) & 15) for t in range(3)]
        srcs = [((lane_i * 3 + t) >> 4) for t in range(3)]
        for g in range(_PW // 16):
            r = [raw_v[pl.ds(g * 48 + j * 16, 16)] for j in range(3)]
            for t in range(3):
                c = [shuf_i(r[j], perms[t]) for j in range(3)]
                out = jnp.where(srcs[t] == 0, c[0],
                                jnp.where(srcs[t] == 1, c[1], c[2]))
                idx_v[pl.ds(t * _PW + g * 16, 16)] = out
        sems = (sem0, sem1)
        lane = lax.broadcasted_iota(jnp.int32, (16,), 0)

        def fire(c, buf):
            return [
                pltpu.async_copy(
                    table_hbm.at[idx_v.at[pl.ds(t * _PW + c * _CH, _CH)]],
                    rows_v.at[buf, t],
                    sems[buf],
                )
                for t in range(3)
            ]

        dnums = lax.GatherDimensionNumbers(
            offset_dims=(), collapsed_slice_dims=(0,), start_index_map=(0,)
        )

        def shuffle(x, perm):
            return lax.gather(
                x, perm[:, None], dnums, slice_sizes=(1,),
                mode=lax.GatherScatterMode.PROMISE_IN_BOUNDS,
            )

        def compute(c, buf):
            def group_fn(g, _):
                def row_fn(r, carry):
                    pv, nv = carry
                    row = g * 16 + r
                    accp = jnp.zeros((16,), jnp.float32)
                    accn = jnp.zeros((16,), jnp.float32)
                    for j in range(D // 16):
                        kv = rows_v[buf, 0, row, pl.ds(j * 16, 16)]
                        accp = accp + kv * rows_v[buf, 1, row, pl.ds(j * 16, 16)]
                        accn = accn + kv * rows_v[buf, 2, row, pl.ds(j * 16, 16)]
                    for s in (8, 4, 2, 1):
                        perm = lane ^ s
                        accp = accp + shuffle(accp, perm)
                        accn = accn + shuffle(accn, perm)
                    pv = jnp.where(lane == r, accp, pv)
                    nv = jnp.where(lane == r, accn, nv)
                    return (pv, nv)

                pv, nv = lax.fori_loop(
                    0, 16, row_fn,
                    (jnp.zeros((16,), jnp.float32), jnp.zeros((16,), jnp.float32)),
                    unroll=2,
                )
                ps_v[pl.ds(c * _CH + g * 16, 16)] = pv
                ns_v[pl.ds(c * _CH + g * 16, 16)] = nv
                return _

            lax.fori_loop(0, _CH // 16, group_fn, 0)

        handles = {0: fire(0, 0)}
        for c in range(_NCH):
            if c + 1 < _NCH:
                handles[c + 1] = fire(c + 1, (c + 1) % 2)
            for h in handles[c]:
                h.wait()
            compute(c, c % 2)

        pltpu.sync_copy(ps_v, out_hbm.at[0, pl.ds(base, _PW)])
        pltpu.sync_copy(ns_v, out_hbm.at[1, pl.ds(base, _PW)])

    return k(table, ts_flat)


def kernel(features, adj, train_set, epoch, W_emb, b_emb, W_cheb, b_cheb):
    del epoch
    bemb2 = b_emb.reshape(1, D)
    bcheb2 = b_cheb.reshape(1, D)

    item_latent = pl.pallas_call(
        _dense_body,
        in_specs=[
            pl.BlockSpec(memory_space=pl.ANY),
            pl.BlockSpec(memory_space=pl.ANY),
            pl.BlockSpec(memory_space=pltpu.MemorySpace.VMEM),
            pl.BlockSpec(memory_space=pltpu.MemorySpace.VMEM),
            pl.BlockSpec(memory_space=pltpu.MemorySpace.VMEM),
            pl.BlockSpec(memory_space=pltpu.MemorySpace.VMEM),
            pl.BlockSpec(memory_space=pltpu.MemorySpace.VMEM),
            pl.BlockSpec(memory_space=pltpu.MemorySpace.VMEM),
        ],
        out_specs=pl.BlockSpec(memory_space=pltpu.MemorySpace.VMEM),
        out_shape=jax.ShapeDtypeStruct((N, D), jnp.float32),
        scratch_shapes=[
            pltpu.VMEM((_FB, F), jnp.float32),
            pltpu.VMEM((_FB, F), jnp.float32),
            pltpu.VMEM((_AB, N), jnp.float32),
            pltpu.VMEM((_AB, N), jnp.float32),
            pltpu.VMEM((_AB, N), jnp.float32),
            pltpu.VMEM((N, N), jnp.bfloat16),
            pltpu.VMEM((N, D), jnp.float32),
            pltpu.VMEM((N, D), jnp.bfloat16),
            pltpu.SemaphoreType.DMA,
            pltpu.SemaphoreType.DMA,
            pltpu.SemaphoreType.DMA,
            pltpu.SemaphoreType.DMA,
            pltpu.SemaphoreType.DMA,
        ],
    )(features, adj, W_emb, bemb2, W_cheb[0], W_cheb[1], W_cheb[2], bcheb2)

    scores = _sc_scores(item_latent, train_set.reshape(3 * B))

    loss, mrr, hr, ndcg = pl.pallas_call(
        _loss_body,
        grid=(1,),
        in_specs=[pl.BlockSpec((2, B), lambda i: (0, 0))],
        out_specs=[
            pl.BlockSpec(memory_space=pltpu.SMEM),
            pl.BlockSpec(memory_space=pltpu.SMEM),
            pl.BlockSpec(memory_space=pltpu.SMEM),
            pl.BlockSpec(memory_space=pltpu.SMEM),
        ],
        out_shape=[
            jax.ShapeDtypeStruct((1, 1), jnp.float32),
            jax.ShapeDtypeStruct((1, 1), jnp.float32),
            jax.ShapeDtypeStruct((1, 1), jnp.float32),
            jax.ShapeDtypeStruct((1, 1), jnp.float32),
        ],
    )(scores)
    return (loss[0, 0], mrr[0, 0], hr[0, 0], ndcg[0, 0])
